# Initial kernel scaffold; baseline (speedup 1.0000x reference)
import jax
import jax.numpy as jnp
from jax.experimental import pallas as pl


def kernel(x_actuators, x_sensors):
    raise NotImplementedError("write your pallas kernel here")


# speedup vs baseline: 9.7530x; 9.7530x over previous
import functools

import jax
import jax.numpy as jnp
from jax import lax
from jax.experimental import pallas as pl

K = 16
RA = 256


def _topk_block(xa_ref, xs_ref, idx_ref, val_ref):
    xa = xa_ref[0]
    xs = xs_ref[0]
    sn = xs.shape[1]

    na = jnp.sqrt(jnp.sum(xa * xa, axis=0))
    nt = jnp.sqrt(jnp.sum(xs * xs, axis=0))

    num = lax.dot_general(xa, xs, (((0,), (0,)), ((), ())),
                          preferred_element_type=jnp.float32)
    den = na[:, None] * nt[None, :]
    sim = num / den
    key = sim * sim

    col = lax.broadcasted_iota(jnp.int32, key.shape, 1)
    for k in range(K):
        m = jnp.max(key, axis=1, keepdims=True)
        hit = key == m
        idx_col = jnp.min(jnp.where(hit, col, sn), axis=1)
        sel = col == idx_col[:, None]
        val_col = jnp.sum(jnp.where(sel, sim, 0.0), axis=1)
        idx_ref[:, k] = idx_col
        val_ref[:, k] = val_col
        key = jnp.where(sel, -1.0, key)


def _topk(x_actuators, x_sensors):
    b, s, a = x_actuators.shape
    sn = x_sensors.shape[2]
    grid = (b, a // RA)
    return pl.pallas_call(
        _topk_block,
        grid=grid,
        in_specs=[
            pl.BlockSpec((1, s, RA), lambda bi, i: (bi, 0, i)),
            pl.BlockSpec((1, s, sn), lambda bi, i: (bi, 0, 0)),
        ],
        out_specs=[
            pl.BlockSpec((RA, K), lambda bi, i: (bi * (a // RA) + i, 0)),
            pl.BlockSpec((RA, K), lambda bi, i: (bi * (a // RA) + i, 0)),
        ],
        out_shape=[
            jax.ShapeDtypeStruct((b * a, K), jnp.int32),
            jax.ShapeDtypeStruct((b * a, K), jnp.float32),
        ],
    )(x_actuators, x_sensors)


def kernel(x_actuators, x_sensors):
    b, s, a = x_actuators.shape
    indices, values = _topk(x_actuators, x_sensors)
    target_nodes = indices.reshape(b, a * K)
    weights = values.reshape(b, a * K)
    source_nodes = jnp.tile(jnp.repeat(jnp.arange(a, dtype=jnp.int32), K)[None, :], (b, 1))
    edges = jnp.stack([source_nodes, target_nodes], axis=1)
    return edges, weights


# speedup vs baseline: 16.4082x; 1.6824x over previous
import functools

import jax
import jax.numpy as jnp
from jax import lax
from jax.experimental import pallas as pl
from jax.experimental.pallas import tpu as pltpu
from jax.experimental.pallas import tpu_sc as plsc

K = 16
RA = 256
NCHUNK = 256
CHUNK = 16


def _tc_block(xa_ref, xs_ref, sim_ref, cid_ref):
    xa = xa_ref[0]
    xs = xs_ref[0]
    sn = xs.shape[1]

    na = jnp.sqrt(jnp.sum(xa * xa, axis=0))
    nt = jnp.sqrt(jnp.sum(xs * xs, axis=0))

    num = lax.dot_general(xa, xs, (((0,), (0,)), ((), ())),
                          preferred_element_type=jnp.float32)
    sim = num / (na[:, None] * nt[None, :])
    sim_ref[...] = sim
    key = sim * sim

    cm = key[:, 0:NCHUNK]
    for j in range(1, CHUNK):
        cm = jnp.maximum(cm, key[:, j * NCHUNK:(j + 1) * NCHUNK])

    col = lax.broadcasted_iota(jnp.int32, cm.shape, 1)
    for k in range(K):
        m = jnp.max(cm, axis=1, keepdims=True)
        hit = cm == m
        cid = jnp.min(jnp.where(hit, col, NCHUNK), axis=1)
        cid_ref[:, k] = cid
        cm = jnp.where(col == cid[:, None], -1.0, cm)


def _tc_stage(x_actuators, x_sensors):
    b, s, a = x_actuators.shape
    sn = x_sensors.shape[2]
    nblk = a // RA
    return pl.pallas_call(
        _tc_block,
        grid=(b, nblk),
        in_specs=[
            pl.BlockSpec((1, s, RA), lambda bi, i: (bi, 0, i)),
            pl.BlockSpec((1, s, sn), lambda bi, i: (bi, 0, 0)),
        ],
        out_specs=[
            pl.BlockSpec((RA, sn), lambda bi, i: (bi * nblk + i, 0)),
            pl.BlockSpec((RA, K), lambda bi, i: (bi * nblk + i, 0)),
        ],
        out_shape=[
            jax.ShapeDtypeStruct((b * a, sn), jnp.float32),
            jax.ShapeDtypeStruct((b * a, K), jnp.int32),
        ],
    )(x_actuators, x_sensors)


NC = 2
NS = 16


def _sc_body(sim_hbm, cid_hbm, idx_hbm, val_hbm,
             row_v, cids_v, oidx_v, oval_v, sem_r, sem_c):
    nc = NC
    nw = nc * NS
    nrows = sim_hbm.shape[0]
    rows_per_w = nrows // nw
    wid = lax.axis_index("s") * nc + lax.axis_index("c")
    base = wid * rows_per_w

    def row_body(r, carry):
        row = base + r
        pltpu.async_copy(sim_hbm.at[row], row_v, sem_r)
        pltpu.async_copy(cid_hbm.at[row], cids_v, sem_c)
        pltpu.make_async_copy(cid_hbm.at[row], cids_v, sem_c).wait()
        pltpu.make_async_copy(sim_hbm.at[row], row_v, sem_r).wait()
        cids = cids_v[...]

        rk = jnp.full((CHUNK,), -1.0, dtype=jnp.float32)
        rc = jnp.zeros((CHUNK,), dtype=jnp.int32)
        for j in range(CHUNK):
            cols = cids + NCHUNK * j
            vals = plsc.load_gather(row_v, [cols])
            keys = vals * vals
            dk, dc = plsc.sort_key_val(keys, cols, descending=True)
            m = dk > rk
            nk = jnp.where(m, dk, rk)
            nco = jnp.where(m, dc, rc)
            rk, rc = plsc.sort_key_val(nk, nco, descending=False)

        fk, fc = plsc.sort_key_val(rk, rc, descending=True)
        fv = plsc.load_gather(row_v, [fc])
        oidx_v[...] = fc
        oval_v[...] = fv
        pltpu.sync_copy(oidx_v, idx_hbm.at[row])
        pltpu.sync_copy(oval_v, val_hbm.at[row])
        return carry

    lax.fori_loop(0, rows_per_w, row_body, 0)


def _sc_stage(sim, cids):
    nrows = sim.shape[0]
    sn = sim.shape[1]
    mesh = plsc.VectorSubcoreMesh(core_axis_name="c", subcore_axis_name="s",
                                  num_cores=NC, num_subcores=NS)
    f = pl.kernel(
        _sc_body,
        out_type=[
            jax.ShapeDtypeStruct((nrows, K), jnp.int32),
            jax.ShapeDtypeStruct((nrows, K), jnp.float32),
        ],
        mesh=mesh,
        compiler_params=pltpu.CompilerParams(needs_layout_passes=False),
        scratch_types=[
            pltpu.VMEM((sn,), jnp.float32),
            pltpu.VMEM((K,), jnp.int32),
            pltpu.VMEM((K,), jnp.int32),
            pltpu.VMEM((K,), jnp.float32),
            pltpu.SemaphoreType.DMA,
            pltpu.SemaphoreType.DMA,
        ],
    )
    return f(sim, cids)


def kernel(x_actuators, x_sensors):
    b, s, a = x_actuators.shape
    sim, cids = _tc_stage(x_actuators, x_sensors)
    indices, values = _sc_stage(sim, cids)
    target_nodes = indices.reshape(b, a * K)
    weights = values.reshape(b, a * K)
    source_nodes = jnp.tile(jnp.repeat(jnp.arange(a, dtype=jnp.int32), K)[None, :], (b, 1))
    edges = jnp.stack([source_nodes, target_nodes], axis=1)
    return edges, weights


# speedup vs baseline: 25.9758x; 1.5831x over previous
import functools

import jax
import jax.numpy as jnp
from jax import lax
from jax.experimental import pallas as pl
from jax.experimental.pallas import tpu as pltpu
from jax.experimental.pallas import tpu_sc as plsc

K = 16
RA = 256
NCHUNK = 256
CHUNK = 16


def _tc_block(xa_ref, xs_ref, sim_ref, cid_ref):
    xa = xa_ref[0]
    xs = xs_ref[0]
    sn = xs.shape[1]

    na = jnp.sqrt(jnp.sum(xa * xa, axis=0))
    nt = jnp.sqrt(jnp.sum(xs * xs, axis=0))

    num = lax.dot_general(xa, xs, (((0,), (0,)), ((), ())),
                          preferred_element_type=jnp.float32)
    sim = num / (na[:, None] * nt[None, :])
    sim_ref[...] = sim
    key = sim * sim

    cm = key[:, 0:NCHUNK]
    for j in range(1, CHUNK):
        cm = jnp.maximum(cm, key[:, j * NCHUNK:(j + 1) * NCHUNK])

    col = lax.broadcasted_iota(jnp.int32, cm.shape, 1)
    for k in range(K):
        m = jnp.max(cm, axis=1, keepdims=True)
        hit = cm == m
        cid = jnp.min(jnp.where(hit, col, NCHUNK), axis=1)
        cid_ref[:, k] = cid
        cm = jnp.where(col == cid[:, None], -1.0, cm)


def _tc_stage(x_actuators, x_sensors):
    b, s, a = x_actuators.shape
    sn = x_sensors.shape[2]
    nblk = a // RA
    return pl.pallas_call(
        _tc_block,
        grid=(b, nblk),
        in_specs=[
            pl.BlockSpec((1, s, RA), lambda bi, i: (bi, 0, i)),
            pl.BlockSpec((1, s, sn), lambda bi, i: (bi, 0, 0)),
        ],
        out_specs=[
            pl.BlockSpec((RA, sn), lambda bi, i: (bi * nblk + i, 0)),
            pl.BlockSpec((RA, K), lambda bi, i: (bi * nblk + i, 0)),
        ],
        out_shape=[
            jax.ShapeDtypeStruct((b * a, sn), jnp.float32),
            jax.ShapeDtypeStruct((b * a, K), jnp.int32),
        ],
    )(x_actuators, x_sensors)


NC = 2
NS = 16


def _row_topk(row_v, cids):
    nodes = []
    for j in range(CHUNK):
        cols = cids + NCHUNK * j
        vals = plsc.load_gather(row_v, [cols])
        keys = vals * vals
        nodes.append(plsc.sort_key_val(keys, cols, descending=True))
    while len(nodes) > 1:
        nxt = []
        for i in range(0, len(nodes), 2):
            ak, ac = nodes[i]
            bk, bc = nodes[i + 1]
            rbk = lax.rev(bk, (0,))
            rbc = lax.rev(bc, (0,))
            m = ak >= rbk
            mk = jnp.where(m, ak, rbk)
            mc = jnp.where(m, ac, rbc)
            nxt.append(plsc.sort_key_val(mk, mc, descending=True))
        nodes = nxt
    fk, fc = nodes[0]
    fv = plsc.load_gather(row_v, [fc])
    return fc, fv


def _sc_body(sim_hbm, cid_hbm, idx_hbm, val_hbm,
             row_v0, row_v1, cids_all, oidx_all, oval_all,
             sem_r0, sem_r1, sem_c):
    nc = NC
    nw = nc * NS
    nrows = sim_hbm.shape[0]
    rows_per_w = nrows // nw
    wid = lax.axis_index("s") * nc + lax.axis_index("c")
    base = wid * rows_per_w
    last = base + rows_per_w - 1

    nk = rows_per_w * K
    pltpu.async_copy(cid_hbm.at[pl.ds(base * K, nk)], cids_all, sem_c)
    pltpu.async_copy(sim_hbm.at[base], row_v0, sem_r0)
    pltpu.make_async_copy(cid_hbm.at[pl.ds(base * K, nk)], cids_all,
                          sem_c).wait()

    def process(r, row_v):
        cids = cids_all[pl.ds(r * K, K)]
        fc, fv = _row_topk(row_v, cids)
        oidx_all[pl.ds(r * K, K)] = fc
        oval_all[pl.ds(r * K, K)] = fv

    def pair_body(p, carry):
        r0 = 2 * p
        pltpu.async_copy(sim_hbm.at[base + r0 + 1], row_v1, sem_r1)
        pltpu.make_async_copy(sim_hbm.at[base + r0], row_v0, sem_r0).wait()
        process(r0, row_v0)
        nxt = jnp.minimum(base + r0 + 2, last)
        pltpu.async_copy(sim_hbm.at[nxt], row_v0, sem_r0)
        pltpu.make_async_copy(sim_hbm.at[base + r0 + 1], row_v1, sem_r1).wait()
        process(r0 + 1, row_v1)
        return carry

    lax.fori_loop(0, rows_per_w // 2, pair_body, 0)
    pltpu.make_async_copy(sim_hbm.at[last], row_v0, sem_r0).wait()

    pltpu.sync_copy(oidx_all, idx_hbm.at[pl.ds(base * K, nk)])
    pltpu.sync_copy(oval_all, val_hbm.at[pl.ds(base * K, nk)])


def _sc_stage(sim, cids):
    nrows = sim.shape[0]
    sn = sim.shape[1]
    nw = NC * NS
    rows_per_w = nrows // nw
    mesh = plsc.VectorSubcoreMesh(core_axis_name="c", subcore_axis_name="s",
                                  num_cores=NC, num_subcores=NS)
    f = pl.kernel(
        _sc_body,
        out_type=[
            jax.ShapeDtypeStruct((nrows * K,), jnp.int32),
            jax.ShapeDtypeStruct((nrows * K,), jnp.float32),
        ],
        mesh=mesh,
        compiler_params=pltpu.CompilerParams(needs_layout_passes=False),
        scratch_types=[
            pltpu.VMEM((sn,), jnp.float32),
            pltpu.VMEM((sn,), jnp.float32),
            pltpu.VMEM((rows_per_w * K,), jnp.int32),
            pltpu.VMEM((rows_per_w * K,), jnp.int32),
            pltpu.VMEM((rows_per_w * K,), jnp.float32),
            pltpu.SemaphoreType.DMA,
            pltpu.SemaphoreType.DMA,
            pltpu.SemaphoreType.DMA,
        ],
    )
    idx, val = f(sim, cids.reshape(-1))
    return idx.reshape(nrows, K), val.reshape(nrows, K)


def kernel(x_actuators, x_sensors):
    b, s, a = x_actuators.shape
    sim, cids = _tc_stage(x_actuators, x_sensors)
    indices, values = _sc_stage(sim, cids)
    target_nodes = indices.reshape(b, a * K)
    weights = values.reshape(b, a * K)
    source_nodes = jnp.tile(jnp.repeat(jnp.arange(a, dtype=jnp.int32), K)[None, :], (b, 1))
    edges = jnp.stack([source_nodes, target_nodes], axis=1)
    return edges, weights


# speedup vs baseline: 30.2120x; 1.1631x over previous
import functools

import jax
import jax.numpy as jnp
from jax import lax
from jax.experimental import pallas as pl
from jax.experimental.pallas import tpu as pltpu
from jax.experimental.pallas import tpu_sc as plsc

K = 16
RA = 256
NCHUNK = 256
CHUNK = 16


def _tc_block(xa_ref, xs_ref, sim_ref, cid_ref):
    xa = xa_ref[0]
    xs = xs_ref[0]
    sn = xs.shape[1]

    na = jnp.sqrt(jnp.sum(xa * xa, axis=0))
    nt = jnp.sqrt(jnp.sum(xs * xs, axis=0))

    num = lax.dot_general(xa, xs, (((0,), (0,)), ((), ())),
                          preferred_element_type=jnp.float32)
    sim = num / (na[:, None] * nt[None, :])
    sim_ref[...] = sim
    key = sim * sim

    cm = key[:, 0:NCHUNK]
    for j in range(1, CHUNK):
        cm = jnp.maximum(cm, key[:, j * NCHUNK:(j + 1) * NCHUNK])

    colf = lax.broadcasted_iota(jnp.int32, cm.shape, 1).astype(jnp.float32)
    for k in range(K):
        m = jnp.max(cm, axis=1, keepdims=True)
        hit = cm == m
        cidf = jnp.min(jnp.where(hit, colf, float(NCHUNK)), axis=1)
        cid_ref[:, k] = cidf.astype(jnp.int32)
        cm = jnp.where(colf == cidf[:, None], -1.0, cm)


def _tc_stage(x_actuators, x_sensors):
    b, s, a = x_actuators.shape
    sn = x_sensors.shape[2]
    nblk = a // RA
    return pl.pallas_call(
        _tc_block,
        grid=(b, nblk),
        in_specs=[
            pl.BlockSpec((1, s, RA), lambda bi, i: (bi, 0, i)),
            pl.BlockSpec((1, s, sn), lambda bi, i: (bi, 0, 0)),
        ],
        out_specs=[
            pl.BlockSpec((RA, sn), lambda bi, i: (bi * nblk + i, 0)),
            pl.BlockSpec((RA, K), lambda bi, i: (bi * nblk + i, 0)),
        ],
        out_shape=[
            jax.ShapeDtypeStruct((b * a, sn), jnp.float32),
            jax.ShapeDtypeStruct((b * a, K), jnp.int32),
        ],
    )(x_actuators, x_sensors)


NC = 2
NS = 16


def _row_topk(row_v, cids):
    nodes = []
    for j in range(CHUNK):
        cols = cids + NCHUNK * j
        vals = plsc.load_gather(row_v, [cols])
        keys = vals * vals
        nodes.append(plsc.sort_key_val(keys, cols, descending=True))
    while len(nodes) > 1:
        nxt = []
        for i in range(0, len(nodes), 2):
            ak, ac = nodes[i]
            bk, bc = nodes[i + 1]
            rbk = lax.rev(bk, (0,))
            rbc = lax.rev(bc, (0,))
            m = ak >= rbk
            mk = jnp.where(m, ak, rbk)
            mc = jnp.where(m, ac, rbc)
            nxt.append(plsc.sort_key_val(mk, mc, descending=True))
        nodes = nxt
    fk, fc = nodes[0]
    fv = plsc.load_gather(row_v, [fc])
    return fc, fv


def _sc_body(sim_hbm, cid_hbm, idx_hbm, val_hbm,
             row_v0, row_v1, cids_all, oidx_all, oval_all,
             sem_r0, sem_r1, sem_c):
    nc = NC
    nw = nc * NS
    nrows = sim_hbm.shape[0]
    rows_per_w = nrows // nw
    wid = lax.axis_index("s") * nc + lax.axis_index("c")
    base = wid * rows_per_w
    last = base + rows_per_w - 1

    nk = rows_per_w * K
    pltpu.async_copy(cid_hbm.at[pl.ds(base * K, nk)], cids_all, sem_c)
    pltpu.async_copy(sim_hbm.at[base], row_v0, sem_r0)
    pltpu.make_async_copy(cid_hbm.at[pl.ds(base * K, nk)], cids_all,
                          sem_c).wait()

    def process(r, row_v):
        cids = cids_all[pl.ds(r * K, K)]
        fc, fv = _row_topk(row_v, cids)
        oidx_all[pl.ds(r * K, K)] = fc
        oval_all[pl.ds(r * K, K)] = fv

    def pair_body(p, carry):
        r0 = 2 * p
        pltpu.async_copy(sim_hbm.at[base + r0 + 1], row_v1, sem_r1)
        pltpu.make_async_copy(sim_hbm.at[base + r0], row_v0, sem_r0).wait()
        process(r0, row_v0)
        nxt = jnp.minimum(base + r0 + 2, last)
        pltpu.async_copy(sim_hbm.at[nxt], row_v0, sem_r0)
        pltpu.make_async_copy(sim_hbm.at[base + r0 + 1], row_v1, sem_r1).wait()
        process(r0 + 1, row_v1)
        return carry

    lax.fori_loop(0, rows_per_w // 2, pair_body, 0)
    pltpu.make_async_copy(sim_hbm.at[last], row_v0, sem_r0).wait()

    pltpu.sync_copy(oidx_all, idx_hbm.at[pl.ds(base * K, nk)])
    pltpu.sync_copy(oval_all, val_hbm.at[pl.ds(base * K, nk)])


def _sc_stage(sim, cids):
    nrows = sim.shape[0]
    sn = sim.shape[1]
    nw = NC * NS
    rows_per_w = nrows // nw
    mesh = plsc.VectorSubcoreMesh(core_axis_name="c", subcore_axis_name="s",
                                  num_cores=NC, num_subcores=NS)
    f = pl.kernel(
        _sc_body,
        out_type=[
            jax.ShapeDtypeStruct((nrows * K,), jnp.int32),
            jax.ShapeDtypeStruct((nrows * K,), jnp.float32),
        ],
        mesh=mesh,
        compiler_params=pltpu.CompilerParams(needs_layout_passes=False),
        scratch_types=[
            pltpu.VMEM((sn,), jnp.float32),
            pltpu.VMEM((sn,), jnp.float32),
            pltpu.VMEM((rows_per_w * K,), jnp.int32),
            pltpu.VMEM((rows_per_w * K,), jnp.int32),
            pltpu.VMEM((rows_per_w * K,), jnp.float32),
            pltpu.SemaphoreType.DMA,
            pltpu.SemaphoreType.DMA,
            pltpu.SemaphoreType.DMA,
        ],
    )
    idx, val = f(sim, cids.reshape(-1))
    return idx.reshape(nrows, K), val.reshape(nrows, K)


def kernel(x_actuators, x_sensors):
    b, s, a = x_actuators.shape
    sim, cids = _tc_stage(x_actuators, x_sensors)
    indices, values = _sc_stage(sim, cids)
    target_nodes = indices.reshape(b, a * K)
    weights = values.reshape(b, a * K)
    source_nodes = jnp.tile(jnp.repeat(jnp.arange(a, dtype=jnp.int32), K)[None, :], (b, 1))
    edges = jnp.stack([source_nodes, target_nodes], axis=1)
    return edges, weights


# speedup vs baseline: 39.8390x; 1.3186x over previous
import functools

import jax
import jax.numpy as jnp
from jax import lax
from jax.experimental import pallas as pl
from jax.experimental.pallas import tpu as pltpu
from jax.experimental.pallas import tpu_sc as plsc

K = 16
RA = 256
NCHUNK = 256
CHUNK = 16


def _tc_block(xa_ref, xs_ref, sim_ref, cid_ref):
    xa = xa_ref[0]
    xs = xs_ref[0]
    sn = xs.shape[1]

    na = jnp.sqrt(jnp.sum(xa * xa, axis=0))
    nt = jnp.sqrt(jnp.sum(xs * xs, axis=0))

    num = lax.dot_general(xa, xs, (((0,), (0,)), ((), ())),
                          preferred_element_type=jnp.float32)
    sim = num / (na[:, None] * nt[None, :])
    sim_ref[...] = sim
    key = sim * sim

    cm = key[:, 0:NCHUNK]
    for j in range(1, CHUNK):
        cm = jnp.maximum(cm, key[:, j * NCHUNK:(j + 1) * NCHUNK])

    colf = lax.broadcasted_iota(jnp.int32, cm.shape, 1).astype(jnp.float32)
    for k in range(K):
        m = jnp.max(cm, axis=1, keepdims=True)
        hit = cm == m
        cidf = jnp.min(jnp.where(hit, colf, float(NCHUNK)), axis=1)
        cid_ref[:, k] = cidf.astype(jnp.int32)
        cm = jnp.where(colf == cidf[:, None], -1.0, cm)


def _tc_stage(x_actuators, x_sensors):
    b, s, a = x_actuators.shape
    sn = x_sensors.shape[2]
    nblk = a // RA
    return pl.pallas_call(
        _tc_block,
        grid=(b, nblk),
        in_specs=[
            pl.BlockSpec((1, s, RA), lambda bi, i: (bi, 0, i)),
            pl.BlockSpec((1, s, sn), lambda bi, i: (bi, 0, 0)),
        ],
        out_specs=[
            pl.BlockSpec((RA, sn), lambda bi, i: (bi * nblk + i, 0)),
            pl.BlockSpec((RA, K), lambda bi, i: (bi * nblk + i, 0)),
        ],
        out_shape=[
            jax.ShapeDtypeStruct((b * a, sn), jnp.float32),
            jax.ShapeDtypeStruct((b * a, K), jnp.int32),
        ],
    )(x_actuators, x_sensors)


def _tc_stage_one(xa, xs):
    s, a = xa.shape
    sn = xs.shape[1]
    nblk = a // RA
    return pl.pallas_call(
        _tc_block,
        grid=(1, nblk),
        in_specs=[
            pl.BlockSpec((1, s, RA), lambda bi, i: (0, 0, i)),
            pl.BlockSpec((1, s, sn), lambda bi, i: (0, 0, 0)),
        ],
        out_specs=[
            pl.BlockSpec((RA, sn), lambda bi, i: (i, 0)),
            pl.BlockSpec((RA, K), lambda bi, i: (i, 0)),
        ],
        out_shape=[
            jax.ShapeDtypeStruct((a, sn), jnp.float32),
            jax.ShapeDtypeStruct((a, K), jnp.int32),
        ],
    )(xa[None], xs[None])


NC = 2
NS = 16


def _row_topk(row_v, cids):
    nodes = []
    for j in range(CHUNK):
        cols = cids + NCHUNK * j
        vals = plsc.load_gather(row_v, [cols])
        keys = vals * vals
        nodes.append(plsc.sort_key_val(keys, cols, descending=True))
    while len(nodes) > 1:
        nxt = []
        for i in range(0, len(nodes), 2):
            ak, ac = nodes[i]
            bk, bc = nodes[i + 1]
            rbk = lax.rev(bk, (0,))
            rbc = lax.rev(bc, (0,))
            m = ak >= rbk
            mk = jnp.where(m, ak, rbk)
            mc = jnp.where(m, ac, rbc)
            nxt.append(plsc.sort_key_val(mk, mc, descending=True))
        nodes = nxt
    fk, fc = nodes[0]
    fv = plsc.load_gather(row_v, [fc])
    return fc, fv


def _sc_body(sim_hbm, cid_hbm, idx_hbm, val_hbm,
             row_v0, row_v1, cids_all, oidx_all, oval_all,
             sem_r0, sem_r1, sem_c):
    nc = NC
    nw = nc * NS
    nrows = sim_hbm.shape[0]
    rows_per_w = nrows // nw
    wid = lax.axis_index("s") * nc + lax.axis_index("c")
    base = wid * rows_per_w
    last = base + rows_per_w - 1

    nk = rows_per_w * K
    pltpu.async_copy(cid_hbm.at[pl.ds(base * K, nk)], cids_all, sem_c)
    pltpu.async_copy(sim_hbm.at[base], row_v0, sem_r0)
    pltpu.make_async_copy(cid_hbm.at[pl.ds(base * K, nk)], cids_all,
                          sem_c).wait()

    def process(r, row_v):
        cids = cids_all[pl.ds(r * K, K)]
        fc, fv = _row_topk(row_v, cids)
        oidx_all[pl.ds(r * K, K)] = fc
        oval_all[pl.ds(r * K, K)] = fv

    def pair_body(p, carry):
        r0 = 2 * p
        pltpu.async_copy(sim_hbm.at[base + r0 + 1], row_v1, sem_r1)
        pltpu.make_async_copy(sim_hbm.at[base + r0], row_v0, sem_r0).wait()
        process(r0, row_v0)
        nxt = jnp.minimum(base + r0 + 2, last)
        pltpu.async_copy(sim_hbm.at[nxt], row_v0, sem_r0)
        pltpu.make_async_copy(sim_hbm.at[base + r0 + 1], row_v1, sem_r1).wait()
        process(r0 + 1, row_v1)
        return carry

    lax.fori_loop(0, rows_per_w // 2, pair_body, 0)
    pltpu.make_async_copy(sim_hbm.at[last], row_v0, sem_r0).wait()

    pltpu.sync_copy(oidx_all, idx_hbm.at[pl.ds(base * K, nk)])
    pltpu.sync_copy(oval_all, val_hbm.at[pl.ds(base * K, nk)])


def _sc_stage(sim, cids):
    nrows = sim.shape[0]
    sn = sim.shape[1]
    nw = NC * NS
    rows_per_w = nrows // nw
    mesh = plsc.VectorSubcoreMesh(core_axis_name="c", subcore_axis_name="s",
                                  num_cores=NC, num_subcores=NS)
    f = pl.kernel(
        _sc_body,
        out_type=[
            jax.ShapeDtypeStruct((nrows * K,), jnp.int32),
            jax.ShapeDtypeStruct((nrows * K,), jnp.float32),
        ],
        mesh=mesh,
        compiler_params=pltpu.CompilerParams(needs_layout_passes=False),
        scratch_types=[
            pltpu.VMEM((sn,), jnp.float32),
            pltpu.VMEM((sn,), jnp.float32),
            pltpu.VMEM((rows_per_w * K,), jnp.int32),
            pltpu.VMEM((rows_per_w * K,), jnp.int32),
            pltpu.VMEM((rows_per_w * K,), jnp.float32),
            pltpu.SemaphoreType.DMA,
            pltpu.SemaphoreType.DMA,
            pltpu.SemaphoreType.DMA,
        ],
    )
    idx, val = f(sim, cids.reshape(-1))
    return idx.reshape(nrows, K), val.reshape(nrows, K)


def kernel(x_actuators, x_sensors):
    b, s, a = x_actuators.shape
    idx_parts, val_parts = [], []
    for bi in range(b):
        sim, cids = _tc_stage_one(x_actuators[bi], x_sensors[bi])
        idx, val = _sc_stage(sim, cids)
        idx_parts.append(idx)
        val_parts.append(val)
    indices = jnp.stack(idx_parts)
    values = jnp.stack(val_parts)
    target_nodes = indices.reshape(b, a * K)
    weights = values.reshape(b, a * K)
    source_nodes = jnp.tile(jnp.repeat(jnp.arange(a, dtype=jnp.int32), K)[None, :], (b, 1))
    edges = jnp.stack([source_nodes, target_nodes], axis=1)
    return edges, weights


# speedup vs baseline: 46.8313x; 1.1755x over previous
import functools

import jax
import jax.numpy as jnp
from jax import lax
from jax.experimental import pallas as pl
from jax.experimental.pallas import tpu as pltpu
from jax.experimental.pallas import tpu_sc as plsc

K = 16
RA = 256
NCHUNK = 256
CHUNK = 16


def _tc_block(xa_ref, xs_ref, sim_ref, cid_ref):
    xa = xa_ref[0]
    xs = xs_ref[0]
    sn = xs.shape[1]

    na = jnp.sqrt(jnp.sum(xa * xa, axis=0))
    nt = jnp.sqrt(jnp.sum(xs * xs, axis=0))

    num = lax.dot_general(xa, xs, (((0,), (0,)), ((), ())),
                          preferred_element_type=jnp.float32)
    sim = num / (na[:, None] * nt[None, :])
    sim_ref[...] = sim
    key = sim * sim

    cm = key[:, 0:NCHUNK]
    for j in range(1, CHUNK):
        cm = jnp.maximum(cm, key[:, j * NCHUNK:(j + 1) * NCHUNK])

    colf = lax.broadcasted_iota(jnp.int32, cm.shape, 1).astype(jnp.float32)
    for k in range(K):
        m = jnp.max(cm, axis=1, keepdims=True)
        hit = cm == m
        cidf = jnp.min(jnp.where(hit, colf, float(NCHUNK)), axis=1)
        cid_ref[:, k] = cidf.astype(jnp.int32)
        cm = jnp.where(colf == cidf[:, None], -1.0, cm)


def _tc_stage(x_actuators, x_sensors):
    b, s, a = x_actuators.shape
    sn = x_sensors.shape[2]
    nblk = a // RA
    return pl.pallas_call(
        _tc_block,
        grid=(b, nblk),
        in_specs=[
            pl.BlockSpec((1, s, RA), lambda bi, i: (bi, 0, i)),
            pl.BlockSpec((1, s, sn), lambda bi, i: (bi, 0, 0)),
        ],
        out_specs=[
            pl.BlockSpec((RA, sn), lambda bi, i: (bi * nblk + i, 0)),
            pl.BlockSpec((RA, K), lambda bi, i: (bi * nblk + i, 0)),
        ],
        out_shape=[
            jax.ShapeDtypeStruct((b * a, sn), jnp.float32),
            jax.ShapeDtypeStruct((b * a, K), jnp.int32),
        ],
    )(x_actuators, x_sensors)


def _tc_stage_one(xa, xs):
    s, a = xa.shape
    sn = xs.shape[1]
    nblk = a // RA
    return pl.pallas_call(
        _tc_block,
        grid=(1, nblk),
        in_specs=[
            pl.BlockSpec((1, s, RA), lambda bi, i: (0, 0, i)),
            pl.BlockSpec((1, s, sn), lambda bi, i: (0, 0, 0)),
        ],
        out_specs=[
            pl.BlockSpec((RA, sn), lambda bi, i: (i, 0)),
            pl.BlockSpec((RA, K), lambda bi, i: (i, 0)),
        ],
        out_shape=[
            jax.ShapeDtypeStruct((a, sn), jnp.float32),
            jax.ShapeDtypeStruct((a, K), jnp.int32),
        ],
    )(xa[None], xs[None])


NC = 2
NS = 16


def _row_topk(row_v, cids):
    def leaf(j):
        cols = cids + NCHUNK * j
        vals = plsc.load_gather(row_v, [cols])
        keys = vals * vals
        return plsc.sort_key_val(keys, cols, descending=True)

    def merge(a, b):
        ak, ac = a
        bk, bc = b
        rbk = lax.rev(bk, (0,))
        rbc = lax.rev(bc, (0,))
        m = ak >= rbk
        mk = jnp.where(m, ak, rbk)
        mc = jnp.where(m, ac, rbc)
        return plsc.sort_key_val(mk, mc, descending=True)

    def subtree(lo):
        m1 = merge(leaf(lo), leaf(lo + 1))
        m2 = merge(leaf(lo + 2), leaf(lo + 3))
        return merge(m1, m2)

    fk, fc = merge(merge(subtree(0), subtree(4)),
                   merge(subtree(8), subtree(12)))
    fv = plsc.load_gather(row_v, [fc])
    return fc, fv


def _sc_body(sim_hbm, cid_hbm, idx_hbm, val_hbm,
             row_a0, row_a1, row_b0, row_b1, cids_all, oidx_all, oval_all,
             sem_a0, sem_a1, sem_b0, sem_b1, sem_c):
    nc = NC
    nw = nc * NS
    nrows = sim_hbm.shape[0]
    rows_per_w = nrows // nw
    wid = lax.axis_index("s") * nc + lax.axis_index("c")
    base = wid * rows_per_w
    last = base + rows_per_w - 1

    nk = rows_per_w * K
    pltpu.async_copy(cid_hbm.at[pl.ds(base * K, nk)], cids_all, sem_c)
    pltpu.async_copy(sim_hbm.at[base], row_a0, sem_a0)
    pltpu.async_copy(sim_hbm.at[base + 1], row_a1, sem_a1)
    pltpu.make_async_copy(cid_hbm.at[pl.ds(base * K, nk)], cids_all,
                          sem_c).wait()

    def process(r, row_v):
        cids = cids_all[pl.ds(r * K, K)]
        fc, fv = _row_topk(row_v, cids)
        oidx_all[pl.ds(r * K, K)] = fc
        oval_all[pl.ds(r * K, K)] = fv

    def do_pair(r, bufs, sems, nxt_bufs, nxt_sems, nxt0, nxt1):
        pltpu.async_copy(sim_hbm.at[nxt0], nxt_bufs[0], nxt_sems[0])
        pltpu.async_copy(sim_hbm.at[nxt1], nxt_bufs[1], nxt_sems[1])
        pltpu.make_async_copy(sim_hbm.at[r], bufs[0], sems[0]).wait()
        pltpu.make_async_copy(sim_hbm.at[r], bufs[1], sems[1]).wait()
        process(r - base, bufs[0])
        process(r - base + 1, bufs[1])

    def quad_body(q, carry):
        r = base + 4 * q
        do_pair(r, (row_a0, row_a1), (sem_a0, sem_a1),
                (row_b0, row_b1), (sem_b0, sem_b1),
                r + 2, r + 3)
        do_pair(r + 2, (row_b0, row_b1), (sem_b0, sem_b1),
                (row_a0, row_a1), (sem_a0, sem_a1),
                jnp.minimum(r + 4, last), jnp.minimum(r + 5, last))
        return carry

    lax.fori_loop(0, rows_per_w // 4, quad_body, 0)
    pltpu.make_async_copy(sim_hbm.at[last], row_a0, sem_a0).wait()
    pltpu.make_async_copy(sim_hbm.at[last], row_a1, sem_a1).wait()

    pltpu.sync_copy(oidx_all, idx_hbm.at[pl.ds(base * K, nk)])
    pltpu.sync_copy(oval_all, val_hbm.at[pl.ds(base * K, nk)])


def _sc_stage(sim, cids):
    nrows = sim.shape[0]
    sn = sim.shape[1]
    nw = NC * NS
    rows_per_w = nrows // nw
    mesh = plsc.VectorSubcoreMesh(core_axis_name="c", subcore_axis_name="s",
                                  num_cores=NC, num_subcores=NS)
    f = pl.kernel(
        _sc_body,
        out_type=[
            jax.ShapeDtypeStruct((nrows * K,), jnp.int32),
            jax.ShapeDtypeStruct((nrows * K,), jnp.float32),
        ],
        mesh=mesh,
        compiler_params=pltpu.CompilerParams(needs_layout_passes=False),
        scratch_types=[
            pltpu.VMEM((sn,), jnp.float32),
            pltpu.VMEM((sn,), jnp.float32),
            pltpu.VMEM((sn,), jnp.float32),
            pltpu.VMEM((sn,), jnp.float32),
            pltpu.VMEM((rows_per_w * K,), jnp.int32),
            pltpu.VMEM((rows_per_w * K,), jnp.int32),
            pltpu.VMEM((rows_per_w * K,), jnp.float32),
            pltpu.SemaphoreType.DMA,
            pltpu.SemaphoreType.DMA,
            pltpu.SemaphoreType.DMA,
            pltpu.SemaphoreType.DMA,
            pltpu.SemaphoreType.DMA,
        ],
    )
    idx, val = f(sim, cids.reshape(-1))
    return idx.reshape(nrows, K), val.reshape(nrows, K)


def kernel(x_actuators, x_sensors):
    b, s, a = x_actuators.shape
    idx_parts, val_parts = [], []
    for bi in range(b):
        sim, cids = _tc_stage_one(x_actuators[bi], x_sensors[bi])
        idx, val = _sc_stage(sim, cids)
        idx_parts.append(idx)
        val_parts.append(val)
    indices = jnp.stack(idx_parts)
    values = jnp.stack(val_parts)
    target_nodes = indices.reshape(b, a * K)
    weights = values.reshape(b, a * K)
    source_nodes = jnp.tile(jnp.repeat(jnp.arange(a, dtype=jnp.int32), K)[None, :], (b, 1))
    edges = jnp.stack([source_nodes, target_nodes], axis=1)
    return edges, weights
